# Initial kernel scaffold; baseline (speedup 1.0000x reference)
import jax
import jax.numpy as jnp
from jax.experimental import pallas as pl


def kernel(x, ptr):
    raise NotImplementedError("write your pallas kernel here")


# speedup vs baseline: 4.8273x; 4.8273x over previous
import functools

import jax
import jax.numpy as jnp
from jax import lax
from jax.experimental import pallas as pl
from jax.experimental.pallas import tpu as pltpu
from jax.experimental.pallas import tpu_sc as plsc

N, D, B = 32768, 128, 16
PAD = 34
NC, NS, L = 2, 16, 16
NW = NC * NS
RPT = N // NW
CH = 256
NCHUNK = RPT // CH
NG = D // L


def _zero_rows(ref, nrows, ngroups):
    z = jnp.zeros((L,), jnp.float32)
    for r in range(nrows):
        for d in range(ngroups):
            ref[r, pl.ds(d * L, L)] = z


def _stage_ptr(ptr_hbm, ptrv):
    nvec = jnp.full((L,), N, jnp.int32)
    ptrv[pl.ds(8, L)] = nvec
    ptrv[pl.ds(24, L)] = nvec
    pltpu.sync_copy(ptr_hbm, ptrv.at[pl.ds(0, 17)])


def _ptr_scalar(ptrv, j):
    return ptrv[pl.ds(j, L)][0]


_mesh = plsc.VectorSubcoreMesh(core_axis_name="c", subcore_axis_name="s")


@functools.partial(
    pl.kernel,
    out_type=jax.ShapeDtypeStruct((NC, B, D), jnp.float32),
    mesh=_mesh,
    scratch_types=[
        pltpu.VMEM((CH, D), jnp.float32),
        pltpu.VMEM((B, D), jnp.float32),
        pltpu.VMEM((B, D), jnp.float32),
        pltpu.VMEM((40,), jnp.int32),
        pltpu.VMEM((B,), jnp.int32),
        pltpu.VMEM_SHARED((B, D), jnp.float32),
    ],
)
def _seg_sums(x_hbm, ptr_hbm, part_hbm, buf, acc, zbuf, ptrv, idv, shared):
    c = lax.axis_index("c")
    s = lax.axis_index("s")

    _stage_ptr(ptr_hbm, ptrv)
    idv[...] = lax.iota(jnp.int32, L)
    _zero_rows(acc, B, NG)

    pj = [_ptr_scalar(ptrv, j) for j in range(B + 1)]
    wid = s * NC + c
    row0 = wid * RPT

    @pl.when(s == 0)
    def _init_shared():
        _zero_rows(zbuf, B, NG)
        pltpu.sync_copy(zbuf, shared)

    zero8 = tuple(jnp.zeros((L,), jnp.float32) for _ in range(NG))

    def chunk_body(i, _):
        base = row0 + i * CH
        pltpu.sync_copy(x_hbm.at[pl.ds(base, CH), :], buf)
        for j in range(B):
            a = jnp.maximum(pj[j], base) - base
            b = jnp.minimum(pj[j + 1], base + CH) - base

            def row_body(r, carry):
                return tuple(carry[d] + buf[r, pl.ds(d * L, L)]
                             for d in range(NG))

            part = lax.fori_loop(a, b, row_body, zero8)
            for d in range(NG):
                acc[j, pl.ds(d * L, L)] += part[d]
        return 0

    lax.fori_loop(0, NCHUNK, chunk_body, 0)

    plsc.subcore_barrier()
    pltpu.sync_copy(acc, shared.at[idv], add=True)
    plsc.subcore_barrier()

    @pl.when(s == 0)
    def _write_part():
        pltpu.sync_copy(shared, part_hbm.at[c])


@functools.partial(
    pl.kernel,
    out_type=jax.ShapeDtypeStruct((B, D), jnp.float32),
    mesh=_mesh,
    scratch_types=[
        pltpu.VMEM((NC, B, D), jnp.float32),
        pltpu.VMEM((B, D), jnp.float32),
        pltpu.VMEM((40,), jnp.int32),
    ],
)
def _combine(part_hbm, ptr_hbm, out_hbm, pbuf, obuf, ptrv):
    c = lax.axis_index("c")
    s = lax.axis_index("s")

    @pl.when(jnp.logical_and(c == 0, s == 0))
    def _do():
        pltpu.sync_copy(part_hbm, pbuf)
        _stage_ptr(ptr_hbm, ptrv)
        for srow in range(B):
            cnt = _ptr_scalar(ptrv, srow + 1) - _ptr_scalar(ptrv, srow)
            cnt_v = jnp.broadcast_to(cnt, (L,)).astype(jnp.float32)
            inv_v = 1.0 / jnp.maximum(cnt_v, 1.0)
            for d in range(NG):
                tot = (pbuf[0, srow, pl.ds(d * L, L)]
                       + pbuf[1, srow, pl.ds(d * L, L)])
                obuf[srow, pl.ds(d * L, L)] = tot * inv_v
        pltpu.sync_copy(obuf, out_hbm)


def kernel(x, ptr):
    part = _seg_sums(x, ptr)
    mean = _combine(part, ptr)
    return jnp.pad(mean, ((0, 0), (0, PAD)))


# speedup vs baseline: 5.3550x; 1.1093x over previous
import functools

import jax
import jax.numpy as jnp
from jax import lax
from jax.experimental import pallas as pl
from jax.experimental.pallas import tpu as pltpu
from jax.experimental.pallas import tpu_sc as plsc

N, D, B = 32768, 128, 16
PAD = 34
NC, NS, L = 2, 16, 16
NW = NC * NS
RPT = N // NW
CH = 256
NCHUNK = RPT // CH
NG = D // L


def _zero_rows(ref, nrows, ngroups):
    z = jnp.zeros((L,), jnp.float32)
    for r in range(nrows):
        for d in range(ngroups):
            ref[r, pl.ds(d * L, L)] = z


def _stage_ptr(ptr_hbm, ptrv):
    nvec = jnp.full((L,), N, jnp.int32)
    ptrv[pl.ds(8, L)] = nvec
    ptrv[pl.ds(24, L)] = nvec
    pltpu.sync_copy(ptr_hbm, ptrv.at[pl.ds(0, 17)])


def _ptr_scalar(ptrv, j):
    return ptrv[pl.ds(j, L)][0]


_mesh = plsc.VectorSubcoreMesh(core_axis_name="c", subcore_axis_name="s")


@functools.partial(
    pl.kernel,
    out_type=jax.ShapeDtypeStruct((NC, B, D), jnp.float32),
    mesh=_mesh,
    scratch_types=[
        pltpu.VMEM((CH, D), jnp.float32),
        pltpu.VMEM((CH, D), jnp.float32),
        pltpu.VMEM((B, D), jnp.float32),
        pltpu.VMEM((B, D), jnp.float32),
        pltpu.VMEM((40,), jnp.int32),
        pltpu.VMEM((B,), jnp.int32),
        pltpu.VMEM_SHARED((B, D), jnp.float32),
        pltpu.SemaphoreType.DMA,
        pltpu.SemaphoreType.DMA,
    ],
)
def _seg_sums(x_hbm, ptr_hbm, part_hbm, buf0, buf1, acc, zbuf, ptrv, idv,
              shared, sem0, sem1):
    c = lax.axis_index("c")
    s = lax.axis_index("s")

    _stage_ptr(ptr_hbm, ptrv)
    idv[...] = lax.iota(jnp.int32, L)
    _zero_rows(acc, B, NG)

    wid = s * NC + c
    row0 = wid * RPT
    bufs, sems = (buf0, buf1), (sem0, sem1)

    @pl.when(s == 0)
    def _init_shared():
        _zero_rows(zbuf, B, NG)
        pltpu.sync_copy(zbuf, shared)

    zero8 = tuple(jnp.zeros((L,), jnp.float32) for _ in range(NG))

    def _dma(chunk, bsel):
        base = row0 + chunk * CH
        return pltpu.make_async_copy(
            x_hbm.at[pl.ds(base, CH), :], bufs[bsel], sems[bsel])

    def _process(buf, base):
        def seg_body(j, _):
            a = jnp.maximum(ptrv[pl.ds(j, L)][0], base) - base
            b = jnp.minimum(ptrv[pl.ds(j + 1, L)][0], base + CH) - base

            @pl.when(b > a)
            def _run():
                def row_body(r, carry):
                    return tuple(carry[d] + buf[r, pl.ds(d * L, L)]
                                 for d in range(NG))

                part = plsc.parallel_loop(a, b, unroll=4,
                                          carry=zero8)(row_body)
                for d in range(NG):
                    acc[j, pl.ds(d * L, L)] += part[d]
            return 0

        lax.fori_loop(0, B, seg_body, 0)

    _dma(0, 0).start()

    def pair_body(i2, _):
        for bsel in range(2):
            chunk = i2 * 2 + bsel

            @pl.when(chunk + 1 < NCHUNK)
            def _prefetch():
                _dma(chunk + 1, 1 - bsel).start()

            _dma(chunk, bsel).wait()
            _process(bufs[bsel], row0 + chunk * CH)
        return 0

    lax.fori_loop(0, NCHUNK // 2, pair_body, 0)

    plsc.subcore_barrier()
    pltpu.sync_copy(acc, shared.at[idv], add=True)
    plsc.subcore_barrier()

    @pl.when(s == 0)
    def _write_part():
        pltpu.sync_copy(shared, part_hbm.at[c])


@functools.partial(
    pl.kernel,
    out_type=jax.ShapeDtypeStruct((B, D), jnp.float32),
    mesh=_mesh,
    scratch_types=[
        pltpu.VMEM((NC, B, D), jnp.float32),
        pltpu.VMEM((B, D), jnp.float32),
        pltpu.VMEM((40,), jnp.int32),
    ],
)
def _combine(part_hbm, ptr_hbm, out_hbm, pbuf, obuf, ptrv):
    c = lax.axis_index("c")
    s = lax.axis_index("s")

    @pl.when(jnp.logical_and(c == 0, s == 0))
    def _do():
        pltpu.sync_copy(part_hbm, pbuf)
        _stage_ptr(ptr_hbm, ptrv)
        for srow in range(B):
            cnt = _ptr_scalar(ptrv, srow + 1) - _ptr_scalar(ptrv, srow)
            cnt_v = jnp.broadcast_to(cnt, (L,)).astype(jnp.float32)
            inv_v = 1.0 / jnp.maximum(cnt_v, 1.0)
            for d in range(NG):
                tot = (pbuf[0, srow, pl.ds(d * L, L)]
                       + pbuf[1, srow, pl.ds(d * L, L)])
                obuf[srow, pl.ds(d * L, L)] = tot * inv_v
        pltpu.sync_copy(obuf, out_hbm)


def kernel(x, ptr):
    part = _seg_sums(x, ptr)
    mean = _combine(part, ptr)
    return jnp.pad(mean, ((0, 0), (0, PAD)))


# speedup vs baseline: 6.2893x; 1.1745x over previous
import functools

import jax
import jax.numpy as jnp
from jax import lax
from jax.experimental import pallas as pl
from jax.experimental.pallas import tpu as pltpu
from jax.experimental.pallas import tpu_sc as plsc

N, D, B = 32768, 128, 16
PAD = 34
NC, NS, L = 2, 16, 16
NW = NC * NS
RPT = N // NW
CH = 256
NCHUNK = RPT // CH
NG = D // L


def _zero_rows(ref, nrows, ngroups):
    z = jnp.zeros((L,), jnp.float32)
    for r in range(nrows):
        for d in range(ngroups):
            ref[r, pl.ds(d * L, L)] = z


def _stage_ptr(ptr_hbm, ptrv):
    nvec = jnp.full((L,), N, jnp.int32)
    ptrv[pl.ds(8, L)] = nvec
    ptrv[pl.ds(24, L)] = nvec
    pltpu.sync_copy(ptr_hbm, ptrv.at[pl.ds(0, 17)])


def _ptr_scalar(ptrv, j):
    return ptrv[pl.ds(j, L)][0]


_mesh = plsc.VectorSubcoreMesh(core_axis_name="c", subcore_axis_name="s")


@functools.partial(
    pl.kernel,
    out_type=jax.ShapeDtypeStruct((NC, B, D), jnp.float32),
    mesh=_mesh,
    scratch_types=[
        pltpu.VMEM((CH, D), jnp.float32),
        pltpu.VMEM((CH, D), jnp.float32),
        pltpu.VMEM((B, D), jnp.float32),
        pltpu.VMEM((B, D), jnp.float32),
        pltpu.VMEM((40,), jnp.int32),
        pltpu.VMEM((B,), jnp.int32),
        pltpu.VMEM_SHARED((B, D), jnp.float32),
        pltpu.SemaphoreType.DMA,
        pltpu.SemaphoreType.DMA,
    ],
)
def _seg_sums(x_hbm, ptr_hbm, part_hbm, buf0, buf1, acc, zbuf, ptrv, idv,
              shared, sem0, sem1):
    c = lax.axis_index("c")
    s = lax.axis_index("s")

    _stage_ptr(ptr_hbm, ptrv)
    idv[...] = lax.iota(jnp.int32, L)
    _zero_rows(acc, B, NG)

    wid = s * NC + c
    row0 = wid * RPT
    bufs, sems = (buf0, buf1), (sem0, sem1)

    @pl.when(s == 0)
    def _init_shared():
        _zero_rows(zbuf, B, NG)
        pltpu.sync_copy(zbuf, shared)

    zero8 = tuple(jnp.zeros((L,), jnp.float32) for _ in range(NG))

    def _dma(chunk, bsel):
        base = row0 + chunk * CH
        return pltpu.make_async_copy(
            x_hbm.at[pl.ds(base, CH), :], bufs[bsel], sems[bsel])

    def _process(buf, base):
        def seg_body(j, _):
            a = jnp.maximum(ptrv[pl.ds(j, L)][0], base) - base
            b = jnp.minimum(ptrv[pl.ds(j + 1, L)][0], base + CH) - base

            @pl.when(b > a)
            def _run():
                def row_body(r, carry):
                    return tuple(carry[d] + buf[r, pl.ds(d * L, L)]
                                 for d in range(NG))

                part = plsc.parallel_loop(a, b, unroll=4,
                                          carry=zero8)(row_body)
                for d in range(NG):
                    acc[j, pl.ds(d * L, L)] += part[d]
            return 0

        lax.fori_loop(0, B, seg_body, 0)

    _dma(0, 0).start()

    def pair_body(i2, _):
        for bsel in range(2):
            chunk = i2 * 2 + bsel

            @pl.when(chunk + 1 < NCHUNK)
            def _prefetch():
                _dma(chunk + 1, 1 - bsel).start()

            _dma(chunk, bsel).wait()
            _process(bufs[bsel], row0 + chunk * CH)
        return 0

    lax.fori_loop(0, NCHUNK // 2, pair_body, 0)

    for j in range(B):
        cnt = _ptr_scalar(ptrv, j + 1) - _ptr_scalar(ptrv, j)
        cnt_v = jnp.broadcast_to(cnt, (L,)).astype(jnp.float32)
        inv_v = 1.0 / jnp.maximum(cnt_v, 1.0)
        for d in range(NG):
            acc[j, pl.ds(d * L, L)] *= inv_v

    plsc.subcore_barrier()
    pltpu.sync_copy(acc, shared.at[idv], add=True)
    plsc.subcore_barrier()

    @pl.when(s == 0)
    def _write_part():
        pltpu.sync_copy(shared, part_hbm.at[c])


def _tc_combine_body(part_ref, out_ref):
    tot = part_ref[0] + part_ref[1]
    pad = jnp.zeros((B, PAD), jnp.float32)
    out_ref[...] = jnp.concatenate([tot, pad], axis=1)


_tc_combine = pl.pallas_call(
    _tc_combine_body,
    out_shape=jax.ShapeDtypeStruct((B, D + PAD), jnp.float32),
)


def kernel(x, ptr):
    part = _seg_sums(x, ptr)
    return _tc_combine(part)
